# Initial kernel scaffold; baseline (speedup 1.0000x reference)
import jax
import jax.numpy as jnp
from jax.experimental import pallas as pl


def kernel(x, edge_index, W1, att_src1, att_dst1, b1, W2, att_src2, att_dst2, b2, W3, att_src3, att_dst3, b3):
    raise NotImplementedError("write your pallas kernel here")


# speedup vs baseline: 31.2112x; 31.2112x over previous
import functools

import jax
import jax.numpy as jnp
import numpy as np
from jax import lax
from jax.experimental import pallas as pl
from jax.experimental.pallas import tpu as pltpu
from jax.experimental.pallas import tpu_sc as plsc

NC = 2
NS = 16
NW = NC * NS
LANES = 16
K_EDGE = 128


def _sc_edge_pass(T, D, src, dst, *, Np, R, H, C, nchunk):
    F = H * C
    rows_tile = Np // NS
    n_out_cp = rows_tile // K_EDGE
    epw = nchunk * K_EDGE

    mesh = plsc.VectorSubcoreMesh(core_axis_name="c", subcore_axis_name="s")

    @functools.partial(
        pl.kernel,
        out_type=jax.ShapeDtypeStruct((NC * Np, R), jnp.float32),
        mesh=mesh,
        compiler_params=pltpu.CompilerParams(
            use_tc_tiling_on_sc=False, needs_layout_passes=False),
        scratch_types=[
            pltpu.VMEM_SHARED((Np, R), jnp.float32),
            pltpu.VMEM((K_EDGE, R), jnp.float32),
            pltpu.VMEM((K_EDGE, 16), jnp.float32),
            pltpu.VMEM((K_EDGE,), jnp.int32),
            pltpu.VMEM((K_EDGE,), jnp.int32),
            pltpu.SemaphoreType.DMA,
            pltpu.SemaphoreType.DMA,
        ],
    )
    def k(T_h, D_h, src_h, dst_h, out_h, acc, G, Dg, si, di, sem, sem2):
        cid = lax.axis_index("c")
        sid = lax.axis_index("s")
        wid = cid * NS + sid
        iot = lax.iota(jnp.int32, LANES)

        def zbody(t, _):
            rowv = jnp.full((LANES,), t, jnp.int32)
            for j in range(R // LANES):
                plsc.store_scatter(G, [rowv, iot + (LANES * j)],
                                   jnp.zeros((LANES,), jnp.float32))
            return 0
        lax.fori_loop(0, K_EDGE, zbody, 0)
        row0 = sid * rows_tile
        for i in range(n_out_cp):
            pltpu.sync_copy(G, acc.at[pl.ds(row0 + i * K_EDGE, K_EDGE)])
        plsc.subcore_barrier()

        def chunk(g, _):
            base = wid * epw + g * K_EDGE
            pltpu.sync_copy(src_h.at[pl.ds(base, K_EDGE)], si)
            pltpu.sync_copy(dst_h.at[pl.ds(base, K_EDGE)], di)
            cp1 = pltpu.async_copy(T_h.at[si], G, sem)
            cp2 = pltpu.async_copy(D_h.at[di], Dg, sem2)
            cp1.wait()
            cp2.wait()
            for q in range(K_EDGE // LANES):
                rows = iot + (LANES * q)
                exs = []
                for h in range(H):
                    colv = jnp.full((LANES,), F + h, jnp.int32)
                    a_s = plsc.load_gather(G, [rows, colv])
                    a_d = plsc.load_gather(
                        Dg, [rows, jnp.full((LANES,), h, jnp.int32)])
                    e = a_s + a_d
                    e = jnp.where(e >= 0.0, e, 0.2 * e)
                    ex = jnp.exp(e)
                    plsc.store_scatter(G, [rows, colv], ex)
                    exs.append(ex)
                for h in range(H):
                    ex = exs[h]
                    def cbody(c, _):
                        colv2 = jnp.full((LANES,), h * C, jnp.int32) + c
                        v = plsc.load_gather(G, [rows, colv2])
                        plsc.store_scatter(G, [rows, colv2], v * ex)
                        return 0
                    lax.fori_loop(0, C, cbody, 0)
            pltpu.sync_copy(G, acc.at[di], add=True)
            return 0
        lax.fori_loop(0, nchunk, chunk, 0)

        plsc.subcore_barrier()
        for i in range(n_out_cp):
            r0 = row0 + i * K_EDGE
            pltpu.sync_copy(acc.at[pl.ds(r0, K_EDGE)],
                            out_h.at[pl.ds(cid * Np + r0, K_EDGE)])

    out = k(T, D, src, dst)
    return out[:Np], out[Np:]


_BM = 256


def _tc_matmul(x, W):
    Np, Kd = x.shape
    Co = W.shape[1]

    def body(xr, wr, yr):
        yr[...] = jnp.dot(xr[...], wr[...],
                          preferred_element_type=jnp.float32)

    return pl.pallas_call(
        body,
        grid=(Np // _BM,),
        in_specs=[
            pl.BlockSpec((_BM, Kd), lambda i: (i, 0)),
            pl.BlockSpec((Kd, Co), lambda i: (0, 0)),
        ],
        out_specs=pl.BlockSpec((_BM, Co), lambda i: (i, 0)),
        out_shape=jax.ShapeDtypeStruct((Np, Co), jnp.float32),
    )(x, W)


def _tc_norm_elu_matmul(A0, A1, ES, Wn, b, F):
    Np, R = A0.shape
    Co = Wn.shape[1]

    def body(a0, a1, es, wn, br, yr):
        s = a0[...] + a1[...]
        den = jnp.dot(s, es[...], preferred_element_type=jnp.float32)
        out = s[:, :F] / (den + 1e-16) + br[...]
        x2 = jnp.where(out > 0.0, out, jnp.exp(out) - 1.0)
        yr[...] = jnp.dot(x2, wn[...], preferred_element_type=jnp.float32)

    return pl.pallas_call(
        body,
        grid=(Np // _BM,),
        in_specs=[
            pl.BlockSpec((_BM, R), lambda i: (i, 0)),
            pl.BlockSpec((_BM, R), lambda i: (i, 0)),
            pl.BlockSpec((R, F), lambda i: (0, 0)),
            pl.BlockSpec((F, Co), lambda i: (0, 0)),
            pl.BlockSpec((1, F), lambda i: (0, 0)),
        ],
        out_specs=pl.BlockSpec((_BM, Co), lambda i: (i, 0)),
        out_shape=jax.ShapeDtypeStruct((Np, Co), jnp.float32),
    )(A0, A1, ES, Wn, b)


def _tc_final(A0, A1, b3):
    Np = A0.shape[0]

    def body(a0, a1, br, yr):
        s = a0[...] + a1[...]
        yr[...] = s[:, 0:1] / (s[:, 1:2] + 1e-16) + br[...]

    return pl.pallas_call(
        body,
        grid=(Np // _BM,),
        in_specs=[
            pl.BlockSpec((_BM, 16), lambda i: (i, 0)),
            pl.BlockSpec((_BM, 16), lambda i: (i, 0)),
            pl.BlockSpec((1, 1), lambda i: (0, 0)),
        ],
        out_specs=pl.BlockSpec((_BM, 1), lambda i: (i, 0)),
        out_shape=jax.ShapeDtypeStruct((Np, 1), jnp.float32),
    )(A0, A1, b3)


def _blockdiag(att):
    H, C = att.shape
    M = jnp.zeros((H * C, H), jnp.float32)
    return M.at[jnp.arange(H * C), jnp.arange(H * C) // C].set(att.reshape(-1))


def _fold_weights(W, att_s, att_d, R):
    Fin, F = W.shape
    H = att_s.shape[0]
    As = W @ _blockdiag(att_s)
    Ad = W @ _blockdiag(att_d)
    z1 = jnp.zeros((Fin, R - F - H), jnp.float32)
    z2 = jnp.zeros((Fin, 16 - H), jnp.float32)
    return jnp.concatenate([W, As, z1, Ad, z2], axis=1)


def _mk_es(R, H, C):
    F = H * C
    M = np.zeros((R, F), np.float32)
    for h in range(H):
        M[F + h, h * C:(h + 1) * C] = 1.0
    return jnp.asarray(M)


def kernel(x, edge_index, W1, att_src1, att_dst1, b1,
           W2, att_src2, att_dst2, b2, W3, att_src3, att_dst3, b3):
    N, Fin = x.shape
    E = edge_index.shape[1]
    Np = ((N + 1 + 255) // 256) * 256
    Etot = E + N
    nchunk = -(-Etot // (NW * K_EDGE))
    Epad = NW * K_EDGE * nchunk

    loops = jnp.arange(N, dtype=edge_index.dtype)
    padv = jnp.full((Epad - Etot,), N, dtype=edge_index.dtype)
    src = jnp.concatenate([edge_index[0], loops, padv])
    dst = jnp.concatenate([edge_index[1], loops, padv])

    xp = jnp.zeros((Np, Fin), jnp.float32).at[:N].set(x)

    Wb1 = _fold_weights(W1, att_src1, att_dst1, 144)
    Y1 = _tc_matmul(xp, Wb1)
    A0, A1 = _sc_edge_pass(Y1[:, :144], Y1[:, 144:160], src, dst,
                           Np=Np, R=144, H=4, C=32, nchunk=nchunk)

    Wb2 = _fold_weights(W2, att_src2, att_dst2, 80)
    Y2 = _tc_norm_elu_matmul(A0, A1, _mk_es(144, 4, 32), Wb2,
                             b1.reshape(1, -1), 128)
    A0, A1 = _sc_edge_pass(Y2[:, :80], Y2[:, 80:96], src, dst,
                           Np=Np, R=80, H=2, C=32, nchunk=nchunk)

    Wb3 = _fold_weights(W3, att_src3, att_dst3, 16)
    Y3 = _tc_norm_elu_matmul(A0, A1, _mk_es(80, 2, 32), Wb3,
                             b2.reshape(1, -1), 64)
    A0, A1 = _sc_edge_pass(Y3[:, :16], Y3[:, 16:32], src, dst,
                           Np=Np, R=16, H=1, C=1, nchunk=nchunk)

    out = _tc_final(A0, A1, b3.reshape(1, 1))
    return out[:N]


# speedup vs baseline: 36.8630x; 1.1811x over previous
import functools

import jax
import jax.numpy as jnp
import numpy as np
from jax import lax
from jax.experimental import pallas as pl
from jax.experimental.pallas import tpu as pltpu
from jax.experimental.pallas import tpu_sc as plsc

NC = 2
NS = 16
NW = NC * NS
LANES = 16
K_EDGE = 96


def _sc_edge_pass(T, D, src, dst, *, Np, R, H, C, nchunk):
    F = H * C
    rows_tile = Np // NS
    n_out_cp = rows_tile // 128
    epw = nchunk * K_EDGE
    assert nchunk % 2 == 0

    mesh = plsc.VectorSubcoreMesh(core_axis_name="c", subcore_axis_name="s")

    @functools.partial(
        pl.kernel,
        out_type=jax.ShapeDtypeStruct((NC * Np, R), jnp.float32),
        mesh=mesh,
        compiler_params=pltpu.CompilerParams(
            use_tc_tiling_on_sc=False, needs_layout_passes=False),
        scratch_types=[
            pltpu.VMEM_SHARED((Np, R), jnp.float32),
            pltpu.VMEM((K_EDGE, R), jnp.float32),
            pltpu.VMEM((K_EDGE, R), jnp.float32),
            pltpu.VMEM((K_EDGE, 16), jnp.float32),
            pltpu.VMEM((K_EDGE, 16), jnp.float32),
            pltpu.VMEM((K_EDGE,), jnp.int32),
            pltpu.VMEM((K_EDGE,), jnp.int32),
            pltpu.VMEM((K_EDGE,), jnp.int32),
            pltpu.VMEM((K_EDGE,), jnp.int32),
            pltpu.SemaphoreType.DMA,
            pltpu.SemaphoreType.DMA,
            pltpu.SemaphoreType.DMA,
            pltpu.SemaphoreType.DMA,
        ],
    )
    def k(T_h, D_h, src_h, dst_h, out_h, acc,
          G0, G1, Dg0, Dg1, si0, si1, di0, di1, sg0, sg1, sd0, sd1):
        cid = lax.axis_index("c")
        sid = lax.axis_index("s")
        wid = cid * NS + sid
        iot = lax.iota(jnp.int32, LANES)
        bufs = ((G0, Dg0, si0, di0, sg0, sd0), (G1, Dg1, si1, di1, sg1, sd1))

        def zbody(t, _):
            rowv = jnp.full((LANES,), t, jnp.int32)
            for j in range(R // LANES):
                plsc.store_scatter(G0, [rowv, iot + (LANES * j)],
                                   jnp.zeros((LANES,), jnp.float32))
            return 0
        lax.fori_loop(0, 64, zbody, 0)
        row0 = sid * rows_tile
        for i in range(rows_tile // 64):
            pltpu.sync_copy(G0.at[pl.ds(0, 64)],
                            acc.at[pl.ds(row0 + i * 64, 64)])
        plsc.subcore_barrier()

        def issue(g, G, Dg, si, di):
            base = wid * epw + g * K_EDGE
            pltpu.sync_copy(src_h.at[pl.ds(base, K_EDGE)], si)
            pltpu.sync_copy(dst_h.at[pl.ds(base, K_EDGE)], di)
            pltpu.async_copy(T_h.at[si], G, bufs[g % 2][4])
            pltpu.async_copy(D_h.at[di], Dg, bufs[g % 2][5])

        issue(0, G0, Dg0, si0, di0)
        issue(1, G1, Dg1, si1, di1)

        def pair(t, _):
            for b, (G, Dg, si, di, sg, sd) in enumerate(bufs):
                g = t * 2 + b
                pltpu.make_async_copy(T_h.at[si], G, sg).wait()
                pltpu.make_async_copy(D_h.at[di], Dg, sd).wait()
                for q in range(K_EDGE // LANES):
                    rows = iot + (LANES * q)
                    exs = []
                    for h in range(H):
                        colv = jnp.full((LANES,), F + h, jnp.int32)
                        a_s = plsc.load_gather(G, [rows, colv])
                        a_d = plsc.load_gather(
                            Dg, [rows, jnp.full((LANES,), h, jnp.int32)])
                        e = a_s + a_d
                        e = jnp.where(e >= 0.0, e, 0.2 * e)
                        ex = jnp.exp(e)
                        plsc.store_scatter(G, [rows, colv], ex)
                        exs.append(ex)
                    for h in range(H):
                        ex = exs[h]
                        def cbody(c, _):
                            colv2 = jnp.full((LANES,), h * C, jnp.int32) + c
                            v = plsc.load_gather(G, [rows, colv2])
                            plsc.store_scatter(G, [rows, colv2], v * ex)
                            return 0
                        lax.fori_loop(0, C, cbody, 0)
                pltpu.sync_copy(G, acc.at[di], add=True)

                @pl.when(g + 2 < nchunk)
                def _():
                    issue_g = g + 2
                    base = wid * epw + issue_g * K_EDGE
                    pltpu.sync_copy(src_h.at[pl.ds(base, K_EDGE)], si)
                    pltpu.sync_copy(dst_h.at[pl.ds(base, K_EDGE)], di)
                    pltpu.async_copy(T_h.at[si], G, sg)
                    pltpu.async_copy(D_h.at[di], Dg, sd)
            return 0
        lax.fori_loop(0, nchunk // 2, pair, 0)

        plsc.subcore_barrier()
        for i in range(n_out_cp):
            r0 = row0 + i * 128
            pltpu.sync_copy(acc.at[pl.ds(r0, 128)],
                            out_h.at[pl.ds(cid * Np + r0, 128)])

    out = k(T, D, src, dst)
    return out[:Np], out[Np:]


_BM = 256


def _tc_matmul(x, W):
    Np, Kd = x.shape
    Co = W.shape[1]

    def body(xr, wr, yr):
        yr[...] = jnp.dot(xr[...], wr[...],
                          preferred_element_type=jnp.float32)

    return pl.pallas_call(
        body,
        grid=(Np // _BM,),
        in_specs=[
            pl.BlockSpec((_BM, Kd), lambda i: (i, 0)),
            pl.BlockSpec((Kd, Co), lambda i: (0, 0)),
        ],
        out_specs=pl.BlockSpec((_BM, Co), lambda i: (i, 0)),
        out_shape=jax.ShapeDtypeStruct((Np, Co), jnp.float32),
    )(x, W)


def _tc_norm_elu_matmul(A0, A1, ES, Wn, b, F):
    Np, R = A0.shape
    Co = Wn.shape[1]

    def body(a0, a1, es, wn, br, yr):
        s = a0[...] + a1[...]
        den = jnp.dot(s, es[...], preferred_element_type=jnp.float32)
        out = s[:, :F] / (den + 1e-16) + br[...]
        x2 = jnp.where(out > 0.0, out, jnp.exp(out) - 1.0)
        yr[...] = jnp.dot(x2, wn[...], preferred_element_type=jnp.float32)

    return pl.pallas_call(
        body,
        grid=(Np // _BM,),
        in_specs=[
            pl.BlockSpec((_BM, R), lambda i: (i, 0)),
            pl.BlockSpec((_BM, R), lambda i: (i, 0)),
            pl.BlockSpec((R, F), lambda i: (0, 0)),
            pl.BlockSpec((F, Co), lambda i: (0, 0)),
            pl.BlockSpec((1, F), lambda i: (0, 0)),
        ],
        out_specs=pl.BlockSpec((_BM, Co), lambda i: (i, 0)),
        out_shape=jax.ShapeDtypeStruct((Np, Co), jnp.float32),
    )(A0, A1, ES, Wn, b)


def _tc_final(A0, A1, b3):
    Np = A0.shape[0]

    def body(a0, a1, br, yr):
        s = a0[...] + a1[...]
        yr[...] = s[:, 0:1] / (s[:, 1:2] + 1e-16) + br[...]

    return pl.pallas_call(
        body,
        grid=(Np // _BM,),
        in_specs=[
            pl.BlockSpec((_BM, 16), lambda i: (i, 0)),
            pl.BlockSpec((_BM, 16), lambda i: (i, 0)),
            pl.BlockSpec((1, 1), lambda i: (0, 0)),
        ],
        out_specs=pl.BlockSpec((_BM, 1), lambda i: (i, 0)),
        out_shape=jax.ShapeDtypeStruct((Np, 1), jnp.float32),
    )(A0, A1, b3)


def _blockdiag(att):
    H, C = att.shape
    M = jnp.zeros((H * C, H), jnp.float32)
    return M.at[jnp.arange(H * C), jnp.arange(H * C) // C].set(att.reshape(-1))


def _fold_weights(W, att_s, att_d, R):
    Fin, F = W.shape
    H = att_s.shape[0]
    As = W @ _blockdiag(att_s)
    Ad = W @ _blockdiag(att_d)
    z1 = jnp.zeros((Fin, R - F - H), jnp.float32)
    z2 = jnp.zeros((Fin, 16 - H), jnp.float32)
    return jnp.concatenate([W, As, z1, Ad, z2], axis=1)


def _mk_es(R, H, C):
    F = H * C
    M = np.zeros((R, F), np.float32)
    for h in range(H):
        M[F + h, h * C:(h + 1) * C] = 1.0
    return jnp.asarray(M)


def kernel(x, edge_index, W1, att_src1, att_dst1, b1,
           W2, att_src2, att_dst2, b2, W3, att_src3, att_dst3, b3):
    N, Fin = x.shape
    E = edge_index.shape[1]
    Np = ((N + 1 + 255) // 256) * 256
    Etot = E + N
    nchunk = -(-Etot // (NW * K_EDGE))
    nchunk += nchunk % 2
    Epad = NW * K_EDGE * nchunk

    loops = jnp.arange(N, dtype=edge_index.dtype)
    padv = jnp.full((Epad - Etot,), N, dtype=edge_index.dtype)
    src = jnp.concatenate([edge_index[0], loops, padv])
    dst = jnp.concatenate([edge_index[1], loops, padv])

    xp = jnp.zeros((Np, Fin), jnp.float32).at[:N].set(x)

    Wb1 = _fold_weights(W1, att_src1, att_dst1, 144)
    Y1 = _tc_matmul(xp, Wb1)
    A0, A1 = _sc_edge_pass(Y1[:, :144], Y1[:, 144:160], src, dst,
                           Np=Np, R=144, H=4, C=32, nchunk=nchunk)

    Wb2 = _fold_weights(W2, att_src2, att_dst2, 80)
    Y2 = _tc_norm_elu_matmul(A0, A1, _mk_es(144, 4, 32), Wb2,
                             b1.reshape(1, -1), 128)
    A0, A1 = _sc_edge_pass(Y2[:, :80], Y2[:, 80:96], src, dst,
                           Np=Np, R=80, H=2, C=32, nchunk=nchunk)

    Wb3 = _fold_weights(W3, att_src3, att_dst3, 16)
    Y3 = _tc_norm_elu_matmul(A0, A1, _mk_es(80, 2, 32), Wb3,
                             b2.reshape(1, -1), 64)
    A0, A1 = _sc_edge_pass(Y3[:, :16], Y3[:, 16:32], src, dst,
                           Np=Np, R=16, H=1, C=1, nchunk=nchunk)

    out = _tc_final(A0, A1, b3.reshape(1, 1))
    return out[:N]
